# baseline (speedup 1.0000x reference)
import functools

import jax
import jax.numpy as jnp
from jax import lax
from jax.experimental import pallas as pl
from jax.experimental.pallas import tpu as pltpu
from jax.experimental.pallas import tpu_sc as plsc

_N = 50000
_E = 1600000
_B = 64
_MIN_SCORE = 0.1

_NC = 2
_NS = 16
_NW = _NC * _NS
_C = 1024
_G = 50
_EW = _C * _G
_E_PAD = _EW * _NW
_NP = 50176
_RT = _NP // _NS
_ZB = 196


def _make_propagate(F):
  mesh = plsc.VectorSubcoreMesh(core_axis_name="c", subcore_axis_name="s")

  @functools.partial(
      pl.kernel,
      mesh=mesh,
      out_type=jax.ShapeDtypeStruct((_NC, _NP, F), jnp.float32),
      compiler_params=pltpu.CompilerParams(use_tc_tiling_on_sc=False),
      scratch_types=[
          pltpu.VMEM((_C,), jnp.int32),
          pltpu.VMEM((_C,), jnp.int32),
          pltpu.VMEM((_C,), jnp.int32),
          pltpu.VMEM((_C, F), jnp.float32),
          pltpu.VMEM((_C, F), jnp.float32),
          pltpu.VMEM((_ZB, F), jnp.float32),
          pltpu.VMEM_SHARED((_NP, F), jnp.float32),
          pltpu.SemaphoreType.DMA,
          pltpu.SemaphoreType.DMA,
      ],
  )
  def prop(table_hbm, row_hbm, col_hbm, out_hbm, ridx0, ridx1, cidx,
           rows0, rows1, zbuf, acc, sem0, sem1):
    cid = lax.axis_index("c")
    sid = lax.axis_index("s")
    wid = sid * _NC + cid
    ridx = (ridx0, ridx1)
    rows = (rows0, rows1)
    sem = (sem0, sem1)

    def zrow(i, _):
      for j in range(F // 16):
        zbuf[i, pl.ds(j * 16, 16)] = jnp.zeros((16,), jnp.float32)
      return 0
    lax.fori_loop(0, _ZB, zrow, 0)
    for b in range(_RT // _ZB):
      pltpu.sync_copy(zbuf, acc.at[pl.ds(sid * _RT + b * _ZB, _ZB)])
    plsc.subcore_barrier()

    base = wid * _EW

    def start_gather(b, g):
      pltpu.sync_copy(row_hbm.at[pl.ds(base + g * _C, _C)], ridx[b])
      pltpu.async_copy(table_hbm.at[ridx[b]], rows[b], sem[b])

    start_gather(0, 0)
    start_gather(1, 1)

    def body(i, _):
      for b in range(2):
        g = 2 * i + b
        pltpu.make_async_copy(table_hbm.at[ridx[b]], rows[b], sem[b]).wait()
        pltpu.sync_copy(col_hbm.at[pl.ds(base + g * _C, _C)], cidx)
        pltpu.sync_copy(rows[b], acc.at[cidx], add=True)

        @pl.when(g + 2 < _G)
        def _():
          start_gather(b, g + 2)
      return 0

    lax.fori_loop(0, _G // 2, body, 0)
    plsc.subcore_barrier()

    pltpu.sync_copy(acc.at[pl.ds(sid * _RT, _RT)],
                    out_hbm.at[cid, pl.ds(sid * _RT, _RT)])

  return prop


_prop16 = _make_propagate(16)


def _pad_table(v):
  out = jnp.zeros((_NP, 16), jnp.float32)
  return lax.dynamic_update_slice(out, v.astype(jnp.float32), (0, 0))


def _propagate(v, row_p, col_p):
  f = v.shape[1]
  outs = []
  for i in range(0, f, 16):
    parts = _prop16(_pad_table(v[:, i:i + 16]), row_p, col_p)
    outs.append((parts[0] + parts[1])[:_N])
  if f <= 16:
    return outs[0][:, :f]
  return jnp.concatenate(outs, axis=1)[:, :f]


def kernel(x, edge_index, batch_index, W_gcn, b_gcn, w_topk,
           pre_W0, pre_b0, pre_g0, pre_be0,
           pre_W1, pre_b1, pre_g1, pre_be1,
           pre_W2, pre_b2, pre_g2, pre_be2,
           high_W0, high_b0, high_g0, high_be0,
           high_W1, high_b1, high_g1, high_be1,
           high_W2, high_b2, high_g2, high_be2,
           lin_W0, lin_b0, lin_W1, lin_b1, fin_W, fin_b):
  n = _N
  row, col = edge_index[0], edge_index[1]
  pad = _E_PAD - _E
  row_p = jnp.concatenate([row, jnp.full((pad,), _N, jnp.int32)])
  col_p = jnp.concatenate([col, jnp.full((pad,), _N, jnp.int32)])

  ones_n = jnp.ones((n, 1), jnp.float32)

  indeg = _propagate(ones_n, row_p, col_p)[:, 0]
  deg = indeg + 1.0
  dinv = jnp.where(deg > 0, deg ** -0.5, 0.0)
  h = (x @ W_gcn)[:, 0]
  h1 = h * dinv
  agg = _propagate(h1[:, None], row_p, col_p)[:, 0]
  attn = (dinv * agg + dinv * dinv * h)[:, None] + b_gcn

  s = (attn * w_topk).sum(-1)
  m = jax.ops.segment_max(s, batch_index, _B)
  e = jnp.exp(s - m[batch_index])
  z = jax.ops.segment_sum(e, batch_index, _B)
  score = e / z[batch_index]
  smax = jax.ops.segment_max(score, batch_index, _B)[batch_index] - 1e-7
  mask = score > jnp.minimum(smax, _MIN_SCORE)
  fmask = mask.astype(jnp.float32)
  cnt = fmask.sum()
  xp = jnp.where(mask[:, None], x * score[:, None], 0.0)

  kdeg = _propagate(fmask[:, None], row_p, col_p)[:, 0]
  degp = fmask * kdeg
  dinvp = jnp.where(degp > 0, degp ** -0.5, 0.0)

  def tag(xx, Ws, b):
    out = xx @ Ws[0]
    hh = xx
    for k in range(1, Ws.shape[0]):
      src = hh * dinvp[:, None]
      hh = _propagate(src, row_p, col_p) * dinvp[:, None]
      out = out + hh @ Ws[k]
    return out + b

  def bn(xx, g, b):
    mu = (xx * fmask[:, None]).sum(0) / cnt
    v = (((xx - mu) ** 2) * fmask[:, None]).sum(0) / cnt
    return g * (xx - mu) / jnp.sqrt(v + 1e-5) + b

  hcur = xp
  for Ws, b, g, be in ((pre_W0, pre_b0, pre_g0, pre_be0),
                       (pre_W1, pre_b1, pre_g1, pre_be1),
                       (pre_W2, pre_b2, pre_g2, pre_be2)):
    hcur = bn(jax.nn.elu(tag(hcur, Ws, b)), g, be)
    hcur = jnp.where(mask[:, None], hcur, 0.0)
  xs = []
  for Ws, b, g, be in ((high_W0, high_b0, high_g0, high_be0),
                       (high_W1, high_b1, high_g1, high_be1),
                       (high_W2, high_b2, high_g2, high_be2)):
    hcur = bn(jax.nn.elu(tag(hcur, Ws, b)), g, be)
    hcur = jnp.where(mask[:, None], hcur, 0.0)
    xs.append(jax.ops.segment_max(
        jnp.where(mask[:, None], hcur, -jnp.inf), batch_index, _B))
  hcat = jnp.concatenate(xs, axis=1)
  hcat = jax.nn.elu(hcat @ lin_W0 + lin_b0)
  hcat = jax.nn.elu(hcat @ lin_W1 + lin_b1)
  out = hcat @ fin_W + fin_b
  return jax.nn.log_softmax(out, axis=1)


# speedup vs baseline: 1.4876x; 1.4876x over previous
import functools

import jax
import jax.numpy as jnp
from jax import lax
from jax.experimental import pallas as pl
from jax.experimental.pallas import tpu as pltpu
from jax.experimental.pallas import tpu_sc as plsc

_N = 50000
_E = 1600000
_B = 64
_MIN_SCORE = 0.1

_NC = 2
_NS = 16
_NW = _NC * _NS
_C = 3136
_G = 16
_EW = _C * _G
_E_PAD = _EW * _NW
_NP = 50176
_RT = _NP // _NS
_ZB = 196


def _make_propagate(F):
  mesh = plsc.VectorSubcoreMesh(core_axis_name="c", subcore_axis_name="s")

  @functools.partial(
      pl.kernel,
      mesh=mesh,
      out_type=jax.ShapeDtypeStruct((_NC, _NP, F), jnp.float32),
      compiler_params=pltpu.CompilerParams(use_tc_tiling_on_sc=False),
      scratch_types=[
          pltpu.VMEM((_C,), jnp.int32),
          pltpu.VMEM((_C,), jnp.int32),
          pltpu.VMEM((_C, F), jnp.float32),
          pltpu.VMEM((_ZB, F), jnp.float32),
          pltpu.VMEM_SHARED((_NP, F), jnp.float32),
          pltpu.SemaphoreType.DMA,
      ],
  )
  def prop(table_hbm, row_hbm, col_hbm, out_hbm, ridx, cidx, rows, zbuf,
           acc, sem):
    cid = lax.axis_index("c")
    sid = lax.axis_index("s")
    wid = sid * _NC + cid

    def zrow(i, _):
      for j in range(F // 16):
        zbuf[i, pl.ds(j * 16, 16)] = jnp.zeros((16,), jnp.float32)
      return 0
    lax.fori_loop(0, _ZB, zrow, 0)
    for b in range(_RT // _ZB):
      pltpu.sync_copy(zbuf, acc.at[pl.ds(sid * _RT + b * _ZB, _ZB)])
    plsc.subcore_barrier()

    base = wid * _EW

    def body(g, _):
      off = base + g * _C
      pltpu.sync_copy(row_hbm.at[pl.ds(off, _C)], ridx)
      pltpu.async_copy(table_hbm.at[ridx], rows, sem).wait()
      pltpu.sync_copy(col_hbm.at[pl.ds(off, _C)], cidx)
      pltpu.sync_copy(rows, acc.at[cidx], add=True)
      return 0

    lax.fori_loop(0, _G, body, 0)
    plsc.subcore_barrier()

    pltpu.sync_copy(acc.at[pl.ds(sid * _RT, _RT)],
                    out_hbm.at[cid, pl.ds(sid * _RT, _RT)])

  return prop


_prop16 = _make_propagate(16)


def _pad_table(v):
  out = jnp.zeros((_NP, 16), jnp.float32)
  return lax.dynamic_update_slice(out, v.astype(jnp.float32), (0, 0))


def _propagate(v, row_p, col_p):
  f = v.shape[1]
  outs = []
  for i in range(0, f, 16):
    parts = _prop16(_pad_table(v[:, i:i + 16]), row_p, col_p)
    outs.append((parts[0] + parts[1])[:_N])
  if f <= 16:
    return outs[0][:, :f]
  return jnp.concatenate(outs, axis=1)[:, :f]


def kernel(x, edge_index, batch_index, W_gcn, b_gcn, w_topk,
           pre_W0, pre_b0, pre_g0, pre_be0,
           pre_W1, pre_b1, pre_g1, pre_be1,
           pre_W2, pre_b2, pre_g2, pre_be2,
           high_W0, high_b0, high_g0, high_be0,
           high_W1, high_b1, high_g1, high_be1,
           high_W2, high_b2, high_g2, high_be2,
           lin_W0, lin_b0, lin_W1, lin_b1, fin_W, fin_b):
  n = _N
  row, col = edge_index[0], edge_index[1]
  pad = _E_PAD - _E
  row_p = jnp.concatenate([row, jnp.full((pad,), _N, jnp.int32)])
  col_p = jnp.concatenate([col, jnp.full((pad,), _N, jnp.int32)])

  ones_n = jnp.ones((n, 1), jnp.float32)

  indeg = _propagate(ones_n, row_p, col_p)[:, 0]
  deg = indeg + 1.0
  dinv = jnp.where(deg > 0, deg ** -0.5, 0.0)
  h = (x @ W_gcn)[:, 0]
  h1 = h * dinv
  agg = _propagate(h1[:, None], row_p, col_p)[:, 0]
  attn = (dinv * agg + dinv * dinv * h)[:, None] + b_gcn

  s = (attn * w_topk).sum(-1)
  m = jax.ops.segment_max(s, batch_index, _B)
  e = jnp.exp(s - m[batch_index])
  z = jax.ops.segment_sum(e, batch_index, _B)
  score = e / z[batch_index]
  smax = jax.ops.segment_max(score, batch_index, _B)[batch_index] - 1e-7
  mask = score > jnp.minimum(smax, _MIN_SCORE)
  fmask = mask.astype(jnp.float32)
  cnt = fmask.sum()
  xp = jnp.where(mask[:, None], x * score[:, None], 0.0)

  kdeg = _propagate(fmask[:, None], row_p, col_p)[:, 0]
  degp = fmask * kdeg
  dinvp = jnp.where(degp > 0, degp ** -0.5, 0.0)

  def tag(xx, Ws, b):
    out = xx @ Ws[0]
    hh = xx
    for k in range(1, Ws.shape[0]):
      src = hh * dinvp[:, None]
      hh = _propagate(src, row_p, col_p) * dinvp[:, None]
      out = out + hh @ Ws[k]
    return out + b

  def bn(xx, g, b):
    mu = (xx * fmask[:, None]).sum(0) / cnt
    v = (((xx - mu) ** 2) * fmask[:, None]).sum(0) / cnt
    return g * (xx - mu) / jnp.sqrt(v + 1e-5) + b

  hcur = xp
  for Ws, b, g, be in ((pre_W0, pre_b0, pre_g0, pre_be0),
                       (pre_W1, pre_b1, pre_g1, pre_be1),
                       (pre_W2, pre_b2, pre_g2, pre_be2)):
    hcur = bn(jax.nn.elu(tag(hcur, Ws, b)), g, be)
    hcur = jnp.where(mask[:, None], hcur, 0.0)
  xs = []
  for Ws, b, g, be in ((high_W0, high_b0, high_g0, high_be0),
                       (high_W1, high_b1, high_g1, high_be1),
                       (high_W2, high_b2, high_g2, high_be2)):
    hcur = bn(jax.nn.elu(tag(hcur, Ws, b)), g, be)
    hcur = jnp.where(mask[:, None], hcur, 0.0)
    xs.append(jax.ops.segment_max(
        jnp.where(mask[:, None], hcur, -jnp.inf), batch_index, _B))
  hcat = jnp.concatenate(xs, axis=1)
  hcat = jax.nn.elu(hcat @ lin_W0 + lin_b0)
  hcat = jax.nn.elu(hcat @ lin_W1 + lin_b1)
  out = hcat @ fin_W + fin_b
  return jax.nn.log_softmax(out, axis=1)
